# baseline (speedup 1.0000x reference)
import functools
import math

import jax
import jax.numpy as jnp
from jax import lax
from jax.experimental import pallas as pl
from jax.experimental.pallas import tpu as pltpu
from jax.experimental.pallas import tpu_sc as plsc

D_MODEL = 64
SCALE = math.sqrt(D_MODEL)

NC = 2
NS = 16
NW = NC * NS
LANES = 16
D_VECS = D_MODEL // LANES

CHUNK = 512


def _make_sc_gather(B: int):
    assert B % (NW * CHUNK) == 0
    b_per_w = B // NW
    steps = b_per_w // CHUNK

    mesh = plsc.VectorSubcoreMesh(
        core_axis_name="c", subcore_axis_name="s",
        num_cores=NC, num_subcores=NS,
    )

    @functools.partial(
        pl.kernel,
        mesh=mesh,
        compiler_params=pltpu.CompilerParams(use_tc_tiling_on_sc=False),
        out_type=jax.ShapeDtypeStruct((B, D_MODEL), jnp.float32),
        scratch_types=[
            pltpu.VMEM((CHUNK,), jnp.int32),
            pltpu.VMEM((CHUNK, D_MODEL), jnp.float32),
            pltpu.SemaphoreType.DMA,
        ],
    )
    def k(idx_hbm, table_hbm, out_hbm, idx_v, rows_v, sem):
        wid = lax.axis_index("s") * NC + lax.axis_index("c")
        base = wid * b_per_w

        def step_body(s, carry):
            off = base + s * CHUNK
            pltpu.sync_copy(idx_hbm.at[pl.ds(off, CHUNK)], idx_v)
            pltpu.async_copy(table_hbm.at[idx_v], rows_v, sem).wait()

            def scale_body(r, c2):
                for c in range(D_VECS):
                    sl = pl.ds(c * LANES, LANES)
                    rows_v[r, sl] = rows_v[r, sl] * SCALE
                return c2

            lax.fori_loop(0, CHUNK, scale_body, 0, unroll=2)
            pltpu.sync_copy(rows_v, out_hbm.at[pl.ds(off, CHUNK)])
            return carry

        lax.fori_loop(0, steps, step_body, 0)

    return k


def kernel(x, table):
    B, T = x.shape
    flat_idx = x.reshape(-1).astype(jnp.int32)
    out = _make_sc_gather(flat_idx.shape[0])(flat_idx, table)
    return out.reshape(B, T, D_MODEL)


# speedup vs baseline: 1.0887x; 1.0887x over previous
import functools
import math

import jax
import jax.numpy as jnp
from jax import lax
from jax.experimental import pallas as pl
from jax.experimental.pallas import tpu as pltpu
from jax.experimental.pallas import tpu_sc as plsc

D_MODEL = 64
SCALE = math.sqrt(D_MODEL)

NC = 2
NS = 16
NW = NC * NS
LANES = 16
D_VECS = D_MODEL // LANES

CHUNK = 256
NBUF = 4


def _make_sc_gather(B: int):
    assert B % (NW * CHUNK * NBUF) == 0
    b_per_w = B // NW
    steps = b_per_w // CHUNK
    groups = steps // NBUF

    mesh = plsc.VectorSubcoreMesh(
        core_axis_name="c", subcore_axis_name="s",
        num_cores=NC, num_subcores=NS,
    )

    @functools.partial(
        pl.kernel,
        mesh=mesh,
        compiler_params=pltpu.CompilerParams(use_tc_tiling_on_sc=False),
        out_type=jax.ShapeDtypeStruct((B, D_MODEL), jnp.float32),
        scratch_types=[
            pltpu.VMEM((b_per_w,), jnp.int32),
            [pltpu.VMEM((CHUNK, D_MODEL), jnp.float32) for _ in range(NBUF)],
            [pltpu.SemaphoreType.DMA for _ in range(NBUF)],
            [pltpu.SemaphoreType.DMA for _ in range(NBUF)],
        ],
    )
    def k(idx_hbm, table_hbm, out_hbm, idx_all, rows, gsem, ssem):
        wid = lax.axis_index("s") * NC + lax.axis_index("c")
        base = wid * b_per_w
        pltpu.sync_copy(idx_hbm.at[pl.ds(base, b_per_w)], idx_all)

        def gather(s, b):
            pltpu.async_copy(
                table_hbm.at[idx_all.at[pl.ds(s * CHUNK, CHUNK)]],
                rows[b], gsem[b])

        def wait_gather(s, b):
            pltpu.make_async_copy(
                table_hbm.at[idx_all.at[pl.ds(s * CHUNK, CHUNK)]],
                rows[b], gsem[b]).wait()

        def scale(b):
            def body(r, c2):
                for c in range(D_VECS):
                    sl = pl.ds(c * LANES, LANES)
                    rows[b][r, sl] = rows[b][r, sl] * SCALE
                return c2
            lax.fori_loop(0, CHUNK, body, 0, unroll=8)

        def store(s, b):
            pltpu.async_copy(
                rows[b], out_hbm.at[pl.ds(base + s * CHUNK, CHUNK)], ssem[b])

        def wait_store(s, b):
            pltpu.make_async_copy(
                rows[b], out_hbm.at[pl.ds(base + s * CHUNK, CHUNK)],
                ssem[b]).wait()

        for b in range(NBUF):
            gather(b, b)

        def group_body(g, carry):
            for b in range(NBUF):
                s = g * NBUF + b
                wait_gather(s, b)
                scale(b)
                store(s, b)
                wait_store(s, b)
                gather(s + NBUF, b)
            return carry

        lax.fori_loop(0, groups - 1, group_body, 0)

        for b in range(NBUF):
            s = (groups - 1) * NBUF + b
            wait_gather(s, b)
            scale(b)
            store(s, b)
        for b in range(NBUF):
            s = (groups - 1) * NBUF + b
            wait_store(s, b)

    return k


def kernel(x, table):
    B, T = x.shape
    flat_idx = x.reshape(-1).astype(jnp.int32)
    out = _make_sc_gather(flat_idx.shape[0])(flat_idx, table)
    return out.reshape(B, T, D_MODEL)
